# baseline (speedup 1.0000x reference)
import functools
import jax
import jax.numpy as jnp
from jax import lax
from jax.experimental import pallas as pl
from jax.experimental.pallas import tpu as pltpu
from jax.experimental.pallas import tpu_sc as plsc

_CENTER_VAR = 0.1
_SIZE_VAR = 0.2
_NB = 2048

_NW = 32


def _softmax_body(conf_ref, scores_ref):
    x = conf_ref[...]
    e = jnp.exp(x)
    s = jnp.sum(e, axis=0, keepdims=True)
    scores_ref[...] = e * (1.0 / s)


def _decode_body(loc_hbm, pri_hbm, out_hbm, l0, l1, p0, p1, o0, o1):
    N = loc_hbm.shape[2]
    wid = lax.axis_index("s") * 2 + lax.axis_index("c")
    b = wid // 2
    pair = wid % 2
    c0 = pair
    c1 = pair + 2
    pltpu.sync_copy(loc_hbm.at[b, c0], l0)
    pltpu.sync_copy(loc_hbm.at[b, c1], l1)
    pltpu.sync_copy(pri_hbm.at[c0], p0)
    pltpu.sync_copy(pri_hbm.at[c1], p1)

    def step(i, _):
        s = pl.ds(i * 16, 16)
        c = l0[s] * _CENTER_VAR * p1[s] + p0[s]
        h = jnp.exp(l1[s] * _SIZE_VAR) * p1[s] * 0.5
        o0[s] = c - h
        o1[s] = c + h
        return 0

    lax.fori_loop(0, N // 16, step, 0)
    pltpu.sync_copy(o0, out_hbm.at[b, c0])
    pltpu.sync_copy(o1, out_hbm.at[b, c1])


def kernel(location_preds, confidence_preds, priors):
    B, N, C = confidence_preds.shape
    conf_t = jnp.transpose(confidence_preds, (2, 0, 1))
    loc_t = jnp.transpose(location_preds, (0, 2, 1))
    pri_t = jnp.transpose(priors, (1, 0))

    grid = pl.cdiv(N, _NB)
    scores_t = pl.pallas_call(
        _softmax_body,
        grid=(grid,),
        in_specs=[pl.BlockSpec((C, B, _NB), lambda i: (0, 0, i))],
        out_specs=pl.BlockSpec((C, B, _NB), lambda i: (0, 0, i)),
        out_shape=jax.ShapeDtypeStruct((C, B, N), jnp.float32),
    )(conf_t)

    decode = functools.partial(
        pl.kernel,
        mesh=plsc.VectorSubcoreMesh(core_axis_name="c", subcore_axis_name="s"),
        out_type=jax.ShapeDtypeStruct((B, 4, N), jnp.float32),
        scratch_types=[pltpu.VMEM((20000,), jnp.float32)] * 6,
    )(_decode_body)
    boxes_t = decode(loc_t, pri_t)

    scores = jnp.transpose(scores_t, (1, 2, 0))
    boxes = jnp.transpose(boxes_t, (0, 2, 1))
    return scores, boxes


# speedup vs baseline: 1.0386x; 1.0386x over previous
import functools
import jax
import jax.numpy as jnp
from jax import lax
from jax.experimental import pallas as pl
from jax.experimental.pallas import tpu as pltpu
from jax.experimental.pallas import tpu_sc as plsc

_CENTER_VAR = 0.1
_SIZE_VAR = 0.2
_NB = 2048

_NW = 32


def _softmax_body(conf_ref, scores_ref):
    x = conf_ref[...]
    e = jnp.exp(x)
    s = jnp.sum(e, axis=0, keepdims=True)
    scores_ref[...] = e * (1.0 / s)


def _decode_body(loc_hbm, pri_hbm, out_hbm, l0, l1, p0, p1, o0, o1):
    N = loc_hbm.shape[2]
    wid = lax.axis_index("s") * 2 + lax.axis_index("c")
    b = wid // 2
    pair = wid % 2
    c0 = pair
    c1 = pair + 2
    pltpu.sync_copy(loc_hbm.at[b, c0], l0)
    pltpu.sync_copy(loc_hbm.at[b, c1], l1)
    pltpu.sync_copy(pri_hbm.at[c0], p0)
    pltpu.sync_copy(pri_hbm.at[c1], p1)

    def step(i, _):
        for k in range(5):
            s = pl.ds(i * 80 + k * 16, 16)
            c = l0[s] * _CENTER_VAR * p1[s] + p0[s]
            h = jnp.exp(l1[s] * _SIZE_VAR) * p1[s] * 0.5
            o0[s] = c - h
            o1[s] = c + h
        return 0

    lax.fori_loop(0, N // 80, step, 0)
    pltpu.sync_copy(o0, out_hbm.at[b, c0])
    pltpu.sync_copy(o1, out_hbm.at[b, c1])


def kernel(location_preds, confidence_preds, priors):
    B, N, C = confidence_preds.shape
    conf_t = jnp.transpose(confidence_preds, (2, 0, 1))
    loc_t = jnp.transpose(location_preds, (0, 2, 1))
    pri_t = jnp.transpose(priors, (1, 0))

    decode = functools.partial(
        pl.kernel,
        mesh=plsc.VectorSubcoreMesh(core_axis_name="c", subcore_axis_name="s"),
        out_type=jax.ShapeDtypeStruct((B, 4, N), jnp.float32),
        scratch_types=[pltpu.VMEM((20000,), jnp.float32)] * 6,
    )(_decode_body)
    boxes_t = decode(loc_t, pri_t)

    grid = pl.cdiv(N, _NB)
    scores_t = pl.pallas_call(
        _softmax_body,
        grid=(grid,),
        in_specs=[pl.BlockSpec((C, B, _NB), lambda i: (0, 0, i))],
        out_specs=pl.BlockSpec((C, B, _NB), lambda i: (0, 0, i)),
        out_shape=jax.ShapeDtypeStruct((C, B, N), jnp.float32),
    )(conf_t)

    scores = jnp.transpose(scores_t, (1, 2, 0))
    boxes = jnp.transpose(boxes_t, (0, 2, 1))
    return scores, boxes


# speedup vs baseline: 1.2499x; 1.2034x over previous
import jax
import jax.numpy as jnp
from jax.experimental import pallas as pl

_CENTER_VAR = 0.1
_SIZE_VAR = 0.2
_NB = 2048


def _body(conf_ref, loc_ref, pri_ref, scores_ref, boxes_ref):
    x = conf_ref[...]
    e = jnp.exp(x)
    s = jnp.sum(e, axis=0, keepdims=True)
    scores_ref[...] = e * (1.0 / s)

    loc = loc_ref[...]
    pr = pri_ref[...]
    lx, ly = loc[:, 0, :], loc[:, 1, :]
    lw, lh = loc[:, 2, :], loc[:, 3, :]
    px, py, pw, ph = pr[0], pr[1], pr[2], pr[3]
    cx = lx * _CENTER_VAR * pw[None] + px[None]
    cy = ly * _CENTER_VAR * ph[None] + py[None]
    hw = jnp.exp(lw * _SIZE_VAR) * pw[None] * 0.5
    hh = jnp.exp(lh * _SIZE_VAR) * ph[None] * 0.5
    boxes_ref[...] = jnp.concatenate(
        [(cx - hw)[:, None, :], (cy - hh)[:, None, :],
         (cx + hw)[:, None, :], (cy + hh)[:, None, :]], axis=1)


def kernel(location_preds, confidence_preds, priors):
    B, N, C = confidence_preds.shape
    conf_t = jnp.transpose(confidence_preds, (2, 0, 1))
    loc_t = jnp.transpose(location_preds, (0, 2, 1))
    pri_t = jnp.transpose(priors, (1, 0))
    grid = pl.cdiv(N, _NB)
    scores_t, boxes_t = pl.pallas_call(
        _body,
        grid=(grid,),
        in_specs=[
            pl.BlockSpec((C, B, _NB), lambda i: (0, 0, i)),
            pl.BlockSpec((B, 4, _NB), lambda i: (0, 0, i)),
            pl.BlockSpec((4, _NB), lambda i: (0, i)),
        ],
        out_specs=[
            pl.BlockSpec((C, B, _NB), lambda i: (0, 0, i)),
            pl.BlockSpec((B, 4, _NB), lambda i: (0, 0, i)),
        ],
        out_shape=[
            jax.ShapeDtypeStruct((C, B, N), jnp.float32),
            jax.ShapeDtypeStruct((B, 4, N), jnp.float32),
        ],
    )(conf_t, loc_t, pri_t)
    scores = jnp.transpose(scores_t, (1, 2, 0))
    boxes = jnp.transpose(boxes_t, (0, 2, 1))
    return scores, boxes


# speedup vs baseline: 1.2600x; 1.0081x over previous
import jax
import jax.numpy as jnp
from jax.experimental import pallas as pl

_CENTER_VAR = 0.1
_SIZE_VAR = 0.2
_NB = 2176


def _body(conf_ref, loc_ref, pri_ref, scores_ref, boxes_ref):
    x = conf_ref[...]
    e = jnp.exp(x)
    s = jnp.sum(e, axis=0, keepdims=True)
    scores_ref[...] = e * (1.0 / s)

    loc = loc_ref[...]
    pr = pri_ref[...]
    lx, ly = loc[:, 0, :], loc[:, 1, :]
    lw, lh = loc[:, 2, :], loc[:, 3, :]
    px, py, pw, ph = pr[0], pr[1], pr[2], pr[3]
    cx = lx * _CENTER_VAR * pw[None] + px[None]
    cy = ly * _CENTER_VAR * ph[None] + py[None]
    hw = jnp.exp(lw * _SIZE_VAR) * pw[None] * 0.5
    hh = jnp.exp(lh * _SIZE_VAR) * ph[None] * 0.5
    boxes_ref[...] = jnp.concatenate(
        [(cx - hw)[:, None, :], (cy - hh)[:, None, :],
         (cx + hw)[:, None, :], (cy + hh)[:, None, :]], axis=1)


def kernel(location_preds, confidence_preds, priors):
    B, N, C = confidence_preds.shape
    conf_t = jnp.transpose(confidence_preds, (2, 0, 1))
    loc_t = jnp.transpose(location_preds, (0, 2, 1))
    pri_t = jnp.transpose(priors, (1, 0))
    grid = pl.cdiv(N, _NB)
    scores_t, boxes_t = pl.pallas_call(
        _body,
        grid=(grid,),
        in_specs=[
            pl.BlockSpec((C, B, _NB), lambda i: (0, 0, i)),
            pl.BlockSpec((B, 4, _NB), lambda i: (0, 0, i)),
            pl.BlockSpec((4, _NB), lambda i: (0, i)),
        ],
        out_specs=[
            pl.BlockSpec((C, B, _NB), lambda i: (0, 0, i)),
            pl.BlockSpec((B, 4, _NB), lambda i: (0, 0, i)),
        ],
        out_shape=[
            jax.ShapeDtypeStruct((C, B, N), jnp.float32),
            jax.ShapeDtypeStruct((B, 4, N), jnp.float32),
        ],
    )(conf_t, loc_t, pri_t)
    scores = jnp.transpose(scores_t, (1, 2, 0))
    boxes = jnp.transpose(boxes_t, (0, 2, 1))
    return scores, boxes
